# Initial kernel scaffold; baseline (speedup 1.0000x reference)
import jax
import jax.numpy as jnp
from jax.experimental import pallas as pl


def kernel(x, edge_index, W1, b1, W2, b2):
    raise NotImplementedError("write your pallas kernel here")


# speedup vs baseline: 29.5379x; 29.5379x over previous
import functools

import jax
import jax.numpy as jnp
from jax import lax
from jax.experimental import pallas as pl
from jax.experimental.pallas import tpu as pltpu
from jax.experimental.pallas import tpu_sc as plsc

N = 10000
E = 320000
D1 = 16
D2P = 48
NC = 40
CHUNK = 80
ROWS = E // CHUNK
RPT = ROWS // 32
NSL = 640
NSL_LAST = N - 15 * NSL

_mesh = plsc.VectorSubcoreMesh(core_axis_name="c", subcore_axis_name="s")


def _acc_slices(s, copy_640, copy_400):
    @pl.when(s < 15)
    def _():
        copy_640(s * NSL)

    @pl.when(s == 15)
    def _():
        copy_400(15 * NSL)


def _deg_body(dst_hbm, aux_hbm, out_hbm, idx_v, ones_v, acc_sh):
    c = lax.axis_index("c")
    s = lax.axis_index("s")
    w = c * 16 + s
    pltpu.sync_copy(dst_hbm.at[w], idx_v)
    pltpu.sync_copy(aux_hbm.at[pl.ds(N, CHUNK)], ones_v)
    _acc_slices(
        s,
        lambda o: pltpu.sync_copy(aux_hbm.at[pl.ds(o, NSL)],
                                  acc_sh.at[pl.ds(o, NSL)]),
        lambda o: pltpu.sync_copy(aux_hbm.at[pl.ds(o, NSL_LAST)],
                                  acc_sh.at[pl.ds(o, NSL_LAST)]),
    )
    plsc.subcore_barrier()

    def body(i, carry):
        pltpu.sync_copy(ones_v, acc_sh.at[idx_v.at[i]], add=True)
        return carry

    lax.fori_loop(0, RPT, body, 0)
    plsc.subcore_barrier()
    _acc_slices(
        s,
        lambda o: pltpu.sync_copy(acc_sh.at[pl.ds(o, NSL)],
                                  out_hbm.at[pl.ds(c * N + o, NSL)]),
        lambda o: pltpu.sync_copy(acc_sh.at[pl.ds(o, NSL_LAST)],
                                  out_hbm.at[pl.ds(c * N + o, NSL_LAST)]),
    )


_SC_PARAMS = pltpu.CompilerParams(use_tc_tiling_on_sc=False)

_deg_call = pl.kernel(
    _deg_body,
    out_type=jax.ShapeDtypeStruct((2 * N, 1), jnp.float32),
    mesh=_mesh,
    compiler_params=_SC_PARAMS,
    scratch_types=[
        pltpu.VMEM((RPT, CHUNK), jnp.int32),
        pltpu.VMEM((CHUNK, 1), jnp.float32),
        pltpu.VMEM_SHARED((N, 1), jnp.float32),
    ],
)


def _scat_body(src_hbm, dst_hbm, g_hbm, out_hbm, idx_s, idx_d, buf_a, buf_b,
               acc_sh, sem_a, sem_b):
    c = lax.axis_index("c")
    s = lax.axis_index("s")
    w = c * 16 + s
    pltpu.sync_copy(src_hbm.at[w], idx_s)
    pltpu.sync_copy(dst_hbm.at[w], idx_d)
    _acc_slices(
        s,
        lambda o: pltpu.sync_copy(g_hbm.at[pl.ds(o, NSL)],
                                  acc_sh.at[pl.ds(o, NSL)]),
        lambda o: pltpu.sync_copy(g_hbm.at[pl.ds(o, NSL_LAST)],
                                  acc_sh.at[pl.ds(o, NSL_LAST)]),
    )
    plsc.subcore_barrier()

    pltpu.async_copy(g_hbm.at[idx_s.at[0]], buf_a, sem_a)

    def body(i, carry):
        ra = 2 * i
        pltpu.make_async_copy(g_hbm.at[idx_s.at[ra]], buf_a, sem_a).wait()
        hb = pltpu.async_copy(g_hbm.at[idx_s.at[ra + 1]], buf_b, sem_b)
        pltpu.sync_copy(buf_a, acc_sh.at[idx_d.at[ra]], add=True)
        hb.wait()
        pltpu.async_copy(g_hbm.at[idx_s.at[ra + 2]], buf_a, sem_a)
        pltpu.sync_copy(buf_b, acc_sh.at[idx_d.at[ra + 1]], add=True)
        return carry

    lax.fori_loop(0, RPT // 2, body, 0)
    pltpu.make_async_copy(g_hbm.at[idx_s.at[RPT - 1]], buf_a, sem_a).wait()
    pltpu.sync_copy(buf_a, acc_sh.at[idx_d.at[RPT - 1]], add=True)
    plsc.subcore_barrier()
    _acc_slices(
        s,
        lambda o: pltpu.sync_copy(acc_sh.at[pl.ds(o, NSL)],
                                  out_hbm.at[pl.ds(c * N + o, NSL)]),
        lambda o: pltpu.sync_copy(acc_sh.at[pl.ds(o, NSL_LAST)],
                                  out_hbm.at[pl.ds(c * N + o, NSL_LAST)]),
    )


def _make_scat(d):
    return pl.kernel(
        _scat_body,
        out_type=jax.ShapeDtypeStruct((2 * N, d), jnp.float32),
        mesh=_mesh,
        compiler_params=_SC_PARAMS,
        scratch_types=[
            pltpu.VMEM((RPT, CHUNK), jnp.int32),
            pltpu.VMEM((RPT, CHUNK), jnp.int32),
            pltpu.VMEM((CHUNK, d), jnp.float32),
            pltpu.VMEM((CHUNK, d), jnp.float32),
            pltpu.VMEM_SHARED((N, d), jnp.float32),
            pltpu.SemaphoreType.DMA,
            pltpu.SemaphoreType.DMA,
        ],
    )


_scat16 = _make_scat(D1)
_scat48 = _make_scat(D2P)


def _tc1_body(deg_ref, x_ref, w1_ref, g1_ref, dinv_ref):
    degp = deg_ref[...]
    deg = degp[:N, :] + degp[N:, :] + 1.0
    dinv = lax.rsqrt(deg)
    h = jnp.dot(x_ref[...], w1_ref[...], preferred_element_type=jnp.float32)
    g1_ref[...] = h * dinv
    dinv_ref[...] = dinv


_tc1 = pl.pallas_call(
    _tc1_body,
    out_shape=(
        jax.ShapeDtypeStruct((N, D1), jnp.float32),
        jax.ShapeDtypeStruct((N, 1), jnp.float32),
    ),
)


def _tc2_body(acc_ref, g1_ref, dinv_ref, w2_ref, b1_ref, g2_ref):
    acc = acc_ref[...]
    g1 = g1_ref[...]
    dinv = dinv_ref[...]
    agg = acc[:N, :] + acc[N:, :] - g1
    z = jnp.maximum(agg * dinv + b1_ref[...], 0.0)
    h2 = jnp.dot(z, w2_ref[...], preferred_element_type=jnp.float32)
    g2_ref[:, :NC] = h2 * dinv
    g2_ref[:, NC:] = jnp.zeros((N, D2P - NC), jnp.float32)


_tc2 = pl.pallas_call(
    _tc2_body,
    out_shape=jax.ShapeDtypeStruct((N, D2P), jnp.float32),
)


def _tc3_body(acc_ref, g2_ref, dinv_ref, b2_ref, out_ref):
    acc = acc_ref[...]
    agg = acc[:N, :] + acc[N:, :] - g2_ref[...]
    o = agg[:, :NC] * dinv_ref[...] + b2_ref[...]
    m = jnp.max(o, axis=1, keepdims=True)
    e = o - m
    lse = jnp.log(jnp.sum(jnp.exp(e), axis=1, keepdims=True))
    out_ref[...] = e - lse


_tc3 = pl.pallas_call(
    _tc3_body,
    out_shape=jax.ShapeDtypeStruct((N, NC), jnp.float32),
)


def kernel(x, edge_index, W1, b1, W2, b2):
    src2d = edge_index[0].reshape(32, RPT, CHUNK)
    dst2d = edge_index[1].reshape(32, RPT, CHUNK)
    aux = jnp.concatenate(
        [jnp.zeros((N, 1), jnp.float32), jnp.ones((CHUNK, 1), jnp.float32)])
    degp = _deg_call(dst2d, aux)
    g1, dinv = _tc1(degp, x, W1)
    acc1 = _scat16(src2d, dst2d, g1)
    g2 = _tc2(acc1, g1, dinv, W2, b1.reshape(1, D1))
    acc2 = _scat48(src2d, dst2d, g2)
    return _tc3(acc2, g2, dinv, b2.reshape(1, NC))


# speedup vs baseline: 48.6635x; 1.6475x over previous
import functools

import jax
import jax.numpy as jnp
from jax import lax
from jax.experimental import pallas as pl
from jax.experimental.pallas import tpu as pltpu
from jax.experimental.pallas import tpu_sc as plsc

N = 10000
E = 320000
D1 = 16
D2P = 48
NC = 40
CHUNK = 80
ROWS = E // CHUNK
RPT = ROWS // 32
NSL = 640
NSL_LAST = N - 15 * NSL

_mesh = plsc.VectorSubcoreMesh(core_axis_name="c", subcore_axis_name="s")


def _acc_slices(s, copy_640, copy_400):
    @pl.when(s < 15)
    def _():
        copy_640(s * NSL)

    @pl.when(s == 15)
    def _():
        copy_400(15 * NSL)


def _deg_body(dst_hbm, aux_hbm, out_hbm, idx_v, ones_v, acc_sh, sem):
    c = lax.axis_index("c")
    s = lax.axis_index("s")
    w = c * 16 + s
    pltpu.sync_copy(dst_hbm.at[w], idx_v)
    pltpu.sync_copy(aux_hbm.at[pl.ds(N, CHUNK)], ones_v)
    _acc_slices(
        s,
        lambda o: pltpu.sync_copy(aux_hbm.at[pl.ds(o, NSL)],
                                  acc_sh.at[pl.ds(o, NSL)]),
        lambda o: pltpu.sync_copy(aux_hbm.at[pl.ds(o, NSL_LAST)],
                                  acc_sh.at[pl.ds(o, NSL_LAST)]),
    )
    plsc.subcore_barrier()

    def body(i, carry):
        pltpu.sync_copy(ones_v, acc_sh.at[idx_v.at[i]], add=True)
        return carry

    lax.fori_loop(0, RPT, body, 0)
    plsc.subcore_barrier()
    _acc_slices(
        s,
        lambda o: pltpu.sync_copy(acc_sh.at[pl.ds(o, NSL)],
                                  out_hbm.at[pl.ds(c * N + o, NSL)]),
        lambda o: pltpu.sync_copy(acc_sh.at[pl.ds(o, NSL_LAST)],
                                  out_hbm.at[pl.ds(c * N + o, NSL_LAST)]),
    )


_SC_PARAMS = pltpu.CompilerParams(use_tc_tiling_on_sc=False)

_deg_call = pl.kernel(
    _deg_body,
    out_type=jax.ShapeDtypeStruct((2 * N, 1), jnp.float32),
    mesh=_mesh,
    compiler_params=_SC_PARAMS,
    scratch_types=[
        pltpu.VMEM((RPT, CHUNK), jnp.int32),
        pltpu.VMEM((CHUNK, 1), jnp.float32),
        pltpu.VMEM_SHARED((N, 1), jnp.float32),
        pltpu.SemaphoreType.DMA,
    ],
)


PF = 5
NBUF = 10
NGRP = (RPT - 2 * PF) // NBUF


def _scat_body(src_hbm, dst_hbm, g_hbm, out_hbm, idx_s, idx_d, buf, acc_sh,
               gsem):
    bufs = [buf.at[pl.ds(b * CHUNK, CHUNK)] for b in range(NBUF)]
    gsems = [gsem.at[b] for b in range(NBUF)]
    c = lax.axis_index("c")
    s = lax.axis_index("s")
    w = c * 16 + s
    pltpu.sync_copy(src_hbm.at[w], idx_s)
    pltpu.sync_copy(dst_hbm.at[w], idx_d)
    _acc_slices(
        s,
        lambda o: pltpu.sync_copy(g_hbm.at[pl.ds(o, NSL)],
                                  acc_sh.at[pl.ds(o, NSL)]),
        lambda o: pltpu.sync_copy(g_hbm.at[pl.ds(o, NSL_LAST)],
                                  acc_sh.at[pl.ds(o, NSL_LAST)]),
    )
    plsc.subcore_barrier()

    def fire_gather(wi, b):
        pltpu.async_copy(g_hbm.at[idx_s.at[wi]], bufs[b], gsems[b])

    def wait_gather(wi, b):
        pltpu.make_async_copy(g_hbm.at[idx_s.at[wi]], bufs[b],
                              gsems[b]).wait()

    def fire_scatter(wi, b):
        pltpu.sync_copy(bufs[b], acc_sh.at[idx_d.at[wi]], add=True)

    def wait_scatter(b):
        pass

    for wi in range(PF):
        fire_gather(wi, wi)
    for wi in range(PF):
        wait_gather(wi, wi)
        fire_scatter(wi, wi)
        fire_gather(wi + PF, wi + PF)

    def body(g, carry):
        w0 = g * NBUF + PF
        for k in range(NBUF):
            b = (PF + k) % NBUF
            bp = k % NBUF
            wi = w0 + k
            wait_gather(wi, b)
            fire_scatter(wi, b)
            wait_scatter(bp)
            fire_gather(wi + PF, bp)
        return carry

    lax.fori_loop(0, NGRP, body, 0)
    base = NGRP * NBUF + PF
    for k in range(PF):
        wi = base + k
        b = wi % NBUF
        bp = (wi + PF) % NBUF
        wait_gather(wi, b)
        fire_scatter(wi, b)
        wait_scatter(bp)
        fire_gather(wi + PF, bp)
    for k in range(PF):
        wi = base + PF + k
        b = wi % NBUF
        wait_gather(wi, b)
        fire_scatter(wi, b)
    for b in range(NBUF):
        wait_scatter(b)
    plsc.subcore_barrier()
    _acc_slices(
        s,
        lambda o: pltpu.sync_copy(acc_sh.at[pl.ds(o, NSL)],
                                  out_hbm.at[pl.ds(c * N + o, NSL)]),
        lambda o: pltpu.sync_copy(acc_sh.at[pl.ds(o, NSL_LAST)],
                                  out_hbm.at[pl.ds(c * N + o, NSL_LAST)]),
    )


def _make_scat(d):
    return pl.kernel(
        _scat_body,
        out_type=jax.ShapeDtypeStruct((2 * N, d), jnp.float32),
        mesh=_mesh,
        compiler_params=_SC_PARAMS,
        scratch_types=[
            pltpu.VMEM((RPT, CHUNK), jnp.int32),
            pltpu.VMEM((RPT, CHUNK), jnp.int32),
            pltpu.VMEM((NBUF * CHUNK, d), jnp.float32),
            pltpu.VMEM_SHARED((N, d), jnp.float32),
            pltpu.SemaphoreType.DMA((NBUF,)),
        ],
    )


_scat16 = _make_scat(D1)
_scat48 = _make_scat(D2P)


def _tc1_body(deg_ref, x_ref, w1_ref, g1_ref, dinv_ref):
    degp = deg_ref[...]
    deg = degp[:N, :] + degp[N:, :] + 1.0
    dinv = lax.rsqrt(deg)
    h = jnp.dot(x_ref[...], w1_ref[...], preferred_element_type=jnp.float32)
    g1_ref[...] = h * dinv
    dinv_ref[...] = dinv


_tc1 = pl.pallas_call(
    _tc1_body,
    out_shape=(
        jax.ShapeDtypeStruct((N, D1), jnp.float32),
        jax.ShapeDtypeStruct((N, 1), jnp.float32),
    ),
)


def _tc2_body(acc_ref, g1_ref, dinv_ref, w2_ref, b1_ref, g2_ref):
    acc = acc_ref[...]
    g1 = g1_ref[...]
    dinv = dinv_ref[...]
    agg = acc[:N, :] + acc[N:, :] - g1
    z = jnp.maximum(agg * dinv + b1_ref[...], 0.0)
    h2 = jnp.dot(z, w2_ref[...], preferred_element_type=jnp.float32)
    g2_ref[:, :NC] = h2 * dinv
    g2_ref[:, NC:] = jnp.zeros((N, D2P - NC), jnp.float32)


_tc2 = pl.pallas_call(
    _tc2_body,
    out_shape=jax.ShapeDtypeStruct((N, D2P), jnp.float32),
)


def _tc3_body(acc_ref, g2_ref, dinv_ref, b2_ref, out_ref):
    acc = acc_ref[...]
    agg = acc[:N, :] + acc[N:, :] - g2_ref[...]
    o = agg[:, :NC] * dinv_ref[...] + b2_ref[...]
    m = jnp.max(o, axis=1, keepdims=True)
    e = o - m
    lse = jnp.log(jnp.sum(jnp.exp(e), axis=1, keepdims=True))
    out_ref[...] = e - lse


_tc3 = pl.pallas_call(
    _tc3_body,
    out_shape=jax.ShapeDtypeStruct((N, NC), jnp.float32),
)


def kernel(x, edge_index, W1, b1, W2, b2):
    src2d = edge_index[0].reshape(32, RPT, CHUNK)
    dst2d = edge_index[1].reshape(32, RPT, CHUNK)
    aux = jnp.concatenate(
        [jnp.zeros((N, 1), jnp.float32), jnp.ones((CHUNK, 1), jnp.float32)])
    degp = _deg_call(dst2d, aux)
    g1, dinv = _tc1(degp, x, W1)
    acc1 = _scat16(src2d, dst2d, g1)
    g2 = _tc2(acc1, g1, dinv, W2, b1.reshape(1, D1))
    acc2 = _scat48(src2d, dst2d, g2)
    return _tc3(acc2, g2, dinv, b2.reshape(1, NC))
